# Initial kernel scaffold; baseline (speedup 1.0000x reference)
import jax
import jax.numpy as jnp
from jax.experimental import pallas as pl


def kernel(input, covariate, edge_index_d, W_d, al_d, ar_d, b_d, edge_index_m, W_m, al_m, ar_m, b_m, edge_index_s, W_s, al_s, ar_s, b_s):
    raise NotImplementedError("write your pallas kernel here")


# speedup vs baseline: 4.5417x; 4.5417x over previous
import jax
import jax.numpy as jnp
from jax import lax
from jax.experimental import pallas as pl
from jax.experimental.pallas import tpu as pltpu

H = 3
HID = 16
N = 400
F = 144
E = 3200
TB = 96


def _dense_h_kernel(xp_ref, cp_ref, w_ref, alm_ref, arm_ref, h_ref, el_ref, er_ref):
    x = xp_ref[...] + cp_ref[...]
    h = jnp.dot(x, w_ref[...], preferred_element_type=jnp.float32)
    h_ref[...] = h
    el_ref[...] = jnp.dot(h, alm_ref[...], preferred_element_type=jnp.float32)
    er_ref[...] = jnp.dot(h, arm_ref[...], preferred_element_type=jnp.float32)


def _alpha_kernel(src_ref, dst_ref, el_ref, er_ref, alpha_ref):
    n_iota = lax.broadcasted_iota(jnp.int32, (E, N), 1)
    gs = (src_ref[...] == n_iota).astype(jnp.float32)
    gd = (dst_ref[...] == n_iota).astype(jnp.float32)
    els = jnp.dot(gs, el_ref[...], preferred_element_type=jnp.float32)
    erd = jnp.dot(gd, er_ref[...], preferred_element_type=jnp.float32)
    e = els + erd
    e = jnp.where(e >= 0, e, 0.2 * e)
    m = jnp.max(e, axis=0, keepdims=True)
    ee = jnp.exp(e - m)
    esum = lax.dot_general(gd, ee, (((0,), (0,)), ((), ())),
                           preferred_element_type=jnp.float32)
    esum_e = jnp.dot(gd, esum, preferred_element_type=jnp.float32)
    alpha_ref[...] = ee / (esum_e + 1e-9)


def _aggregate_kernel(src_ref, dst_ref, alpha_ref, h_ref, out_ref):
    n_iota = lax.broadcasted_iota(jnp.int32, (E, N), 1)
    gs = (src_ref[...] == n_iota).astype(jnp.bfloat16)
    gd = (dst_ref[...] == n_iota).astype(jnp.bfloat16)
    hb = h_ref[...].astype(jnp.bfloat16)
    z = jnp.dot(gs, hb, preferred_element_type=jnp.float32)
    k_iota = lax.broadcasted_iota(jnp.int32, (48, 768), 0)
    c_iota = lax.broadcasted_iota(jnp.int32, (48, 768), 1)
    ex = (c_iota // HID == k_iota).astype(jnp.float32)
    ae = jnp.dot(alpha_ref[0], ex, preferred_element_type=jnp.float32)
    zz = (z * ae).astype(jnp.bfloat16)
    out_ref[...] = lax.dot_general(gd, zz, (((0,), (0,)), ((), ())),
                                   preferred_element_type=jnp.float32)


def _attn_kernel(cov_ref, out_ref, acc_ref):
    t = pl.program_id(1)

    @pl.when(t == 0)
    def _():
        c = cov_ref[0]
        a = lax.dot_general(c, c, (((1,), (1,)), ((), ())),
                            preferred_element_type=jnp.float32)
        m = jnp.max(a, axis=1, keepdims=True)
        p = jnp.exp(a - m)
        acc_ref[...] = p / jnp.sum(p, axis=1, keepdims=True)

    out_ref[0] = acc_ref[...]


def _expand_al(al, g):
    flat = al.reshape(48)
    rows = jnp.arange(48) // HID + g * H
    onehot = (jnp.arange(16)[None, :] == rows[:, None]).astype(jnp.float32)
    return flat[:, None] * onehot


def _gat_edge(src2, dst2, el_g, er_g, h_g):
    alpha = pl.pallas_call(
        _alpha_kernel,
        out_shape=jax.ShapeDtypeStruct((E, TB * H), jnp.float32),
        in_specs=[
            pl.BlockSpec((E, 1), lambda: (0, 0)),
            pl.BlockSpec((E, 1), lambda: (0, 0)),
            pl.BlockSpec((N, TB * H), lambda: (0, 0)),
            pl.BlockSpec((N, TB * H), lambda: (0, 0)),
        ],
        out_specs=pl.BlockSpec((E, TB * H), lambda: (0, 0)),
    )(src2, dst2, el_g, er_g)

    alpha6 = alpha.reshape(E, 6, 48).transpose(1, 0, 2)
    out = pl.pallas_call(
        _aggregate_kernel,
        grid=(6,),
        out_shape=jax.ShapeDtypeStruct((N, TB * 48), jnp.float32),
        in_specs=[
            pl.BlockSpec((E, 1), lambda j: (0, 0)),
            pl.BlockSpec((E, 1), lambda j: (0, 0)),
            pl.BlockSpec((1, E, 48), lambda j: (j, 0, 0)),
            pl.BlockSpec((N, 768), lambda j: (0, j)),
        ],
        out_specs=pl.BlockSpec((N, 768), lambda j: (0, j)),
    )(src2, dst2, alpha6, h_g)
    return out


def kernel(input, covariate, edge_index_d, W_d, al_d, ar_d, b_d,
           edge_index_m, W_m, al_m, ar_m, b_m,
           edge_index_s, W_s, al_s, ar_s, b_s):
    batch, T = input.shape[0], input.shape[1]
    xp = jnp.transpose(input, (2, 1, 0, 3)).reshape(N * TB, F)
    cp = jnp.transpose(covariate, (2, 1, 0, 3)).reshape(N * TB, F)
    Wcat = jnp.concatenate([W_d, W_m, W_s], axis=1)
    ALcat = jnp.concatenate([_expand_al(al_d, 0), _expand_al(al_m, 1),
                             _expand_al(al_s, 2)], axis=0)
    ARcat = jnp.concatenate([_expand_al(ar_d, 0), _expand_al(ar_m, 1),
                             _expand_al(ar_s, 2)], axis=0)

    blk = 768
    grid_a = (N * TB) // blk
    h_all, el_all, er_all = pl.pallas_call(
        _dense_h_kernel,
        grid=(grid_a,),
        out_shape=[
            jax.ShapeDtypeStruct((N * TB, F), jnp.float32),
            jax.ShapeDtypeStruct((N * TB, 16), jnp.float32),
            jax.ShapeDtypeStruct((N * TB, 16), jnp.float32),
        ],
        in_specs=[
            pl.BlockSpec((blk, F), lambda i: (i, 0)),
            pl.BlockSpec((blk, F), lambda i: (i, 0)),
            pl.BlockSpec((F, F), lambda i: (0, 0)),
            pl.BlockSpec((F, 16), lambda i: (0, 0)),
            pl.BlockSpec((F, 16), lambda i: (0, 0)),
        ],
        out_specs=[
            pl.BlockSpec((blk, F), lambda i: (i, 0)),
            pl.BlockSpec((blk, 16), lambda i: (i, 0)),
            pl.BlockSpec((blk, 16), lambda i: (i, 0)),
        ],
    )(xp, cp, Wcat, ALcat, ARcat)

    h3 = h_all.reshape(N, TB, F)
    el3 = el_all.reshape(N, TB, 16)
    er3 = er_all.reshape(N, TB, 16)

    outs = []
    for g, (edge_index, b) in enumerate([(edge_index_d, b_d),
                                         (edge_index_m, b_m),
                                         (edge_index_s, b_s)]):
        src2 = edge_index[0].astype(jnp.int32).reshape(E, 1)
        dst2 = edge_index[1].astype(jnp.int32).reshape(E, 1)
        h_g = h3[:, :, g * 48:(g + 1) * 48].reshape(N, TB * 48)
        el_g = el3[:, :, g * H:(g + 1) * H].reshape(N, TB * H)
        er_g = er3[:, :, g * H:(g + 1) * H].reshape(N, TB * H)
        out_g = _gat_edge(src2, dst2, el_g, er_g, h_g)
        outs.append(out_g.reshape(N, TB, H, HID) + b.reshape(1, 1, H, HID))

    x_attn = jnp.concatenate(outs, axis=-1)
    x_attn = jnp.transpose(x_attn, (1, 0, 2, 3)).reshape(batch, T, N, F)
    out = input + x_attn

    cov0 = covariate[:, 0]
    attn = pl.pallas_call(
        _attn_kernel,
        grid=(batch, T),
        out_shape=jax.ShapeDtypeStruct((batch * T, N, N), jnp.float32),
        in_specs=[pl.BlockSpec((1, N, F), lambda b, t: (b, 0, 0))],
        out_specs=pl.BlockSpec((1, N, N), lambda b, t: (t * batch + b, 0, 0)),
        scratch_shapes=[pltpu.VMEM((N, N), jnp.float32)],
    )(cov0)
    return out, attn
